# baseline (speedup 1.0000x reference)
import functools

import jax
import jax.numpy as jnp
from jax import lax
from jax.experimental import pallas as pl
from jax.experimental.pallas import tpu as pltpu
from jax.experimental.pallas import tpu_sc as plsc

NUM_EMBEDDINGS = 1024
EMBEDDING_DIM = 32
BETA = 0.25
N = 65536
BLOCK = 2048
GRID = N // BLOCK

NW = 32
ROWS_PER_W = N // NW
CH = 128
NCH = ROWS_PER_W // CH


def _argmin_kernel(z_ref, w_ref, onehot_ref, idx_ref, counts_ref):
    z = z_ref[...]
    w = w_ref[...]

    zn = jnp.sum(z * z, axis=1, keepdims=True)
    wn = jnp.sum(w * w, axis=1)
    s = jnp.dot((z * -2.0).astype(jnp.bfloat16), w.astype(jnp.bfloat16).T,
                preferred_element_type=jnp.float32)
    dist = (zn + wn) + s

    iota = jax.lax.broadcasted_iota(
        jnp.int32, (BLOCK, NUM_EMBEDDINGS), 1).astype(jnp.float32)
    minval = jnp.min(dist, axis=1, keepdims=True)
    idxf = jnp.min(jnp.where(dist == minval, iota, float(NUM_EMBEDDINGS)),
                   axis=1, keepdims=True)
    onehot = (iota == idxf).astype(jnp.float32)
    onehot_ref[...] = onehot
    idx_ref[...] = idxf.astype(jnp.int32)
    ones_row = jnp.ones((1, BLOCK), dtype=jnp.bfloat16)
    counts_ref[...] = jnp.dot(ones_row, onehot.astype(jnp.bfloat16),
                              preferred_element_type=jnp.float32)[None]


def _bf16_round(x):
    c = x * 65537.0
    return c - (c - x)


def _sc_lookup_kernel(z_hbm, w_hbm, idx_hbm, zq_hbm, loss_hbm,
                      idx_v, zrows, grows, loss_v, sem):
    wid = lax.axis_index("s") * 2 + lax.axis_index("c")

    def chunk_body(c, acc):
        base = wid * ROWS_PER_W + c * CH
        pltpu.sync_copy(idx_hbm.at[pl.ds(base, CH)], idx_v)
        pltpu.async_copy(w_hbm.at[idx_v], grows, sem).wait()
        pltpu.sync_copy(z_hbm.at[pl.ds(base, CH)], zrows)

        def row_body(r, a):
            for h in range(2):
                wv = grows[r, pl.ds(h * 16, 16)]
                zv = zrows[r, pl.ds(h * 16, 16)]
                zq = _bf16_round(wv)
                zrows[r, pl.ds(h * 16, 16)] = zv + (zq - zv)
                d = zq - zv
                a = a + d * d
            return a
        acc = lax.fori_loop(0, CH, row_body, acc)

        pltpu.sync_copy(zrows, zq_hbm.at[pl.ds(base, CH)])
        return acc

    loss_acc = lax.fori_loop(0, NCH, chunk_body, jnp.zeros((16,), jnp.float32))
    loss_v[...] = loss_acc
    pltpu.sync_copy(loss_v, loss_hbm.at[wid])


def _finish_kernel(counts_ref, loss_ref, out_loss_ref, out_perp_ref):
    counts = jnp.sum(counts_ref[...], axis=0)
    e_mean = counts / float(N)
    out_perp_ref[...] = jnp.exp(
        -jnp.sum(e_mean * jnp.log(e_mean + 1e-10)))[None, None]
    loss_sum = jnp.sum(loss_ref[...])
    out_loss_ref[...] = (loss_sum * ((1.0 + BETA) / float(N * EMBEDDING_DIM))
                         )[None, None]


def kernel(z, W):
    onehot, idx, counts_p = pl.pallas_call(
        _argmin_kernel,
        grid=(GRID,),
        in_specs=[
            pl.BlockSpec((BLOCK, EMBEDDING_DIM), lambda i: (i, 0)),
            pl.BlockSpec((NUM_EMBEDDINGS, EMBEDDING_DIM), lambda i: (0, 0)),
        ],
        out_specs=(
            pl.BlockSpec((BLOCK, NUM_EMBEDDINGS), lambda i: (i, 0)),
            pl.BlockSpec((BLOCK, 1), lambda i: (i, 0)),
            pl.BlockSpec((1, 1, NUM_EMBEDDINGS), lambda i: (i, 0, 0)),
        ),
        out_shape=(
            jax.ShapeDtypeStruct((N, NUM_EMBEDDINGS), jnp.float32),
            jax.ShapeDtypeStruct((N, 1), jnp.int32),
            jax.ShapeDtypeStruct((GRID, 1, NUM_EMBEDDINGS), jnp.float32),
        ),
        compiler_params=pltpu.CompilerParams(
            dimension_semantics=("parallel",),
        ),
    )(z, W)

    sc = functools.partial(
        pl.kernel,
        out_type=(
            jax.ShapeDtypeStruct((N, EMBEDDING_DIM), jnp.float32),
            jax.ShapeDtypeStruct((NW, 16), jnp.float32),
        ),
        mesh=plsc.VectorSubcoreMesh(core_axis_name="c", subcore_axis_name="s"),
        scratch_types=[
            pltpu.VMEM((CH,), jnp.int32),
            pltpu.VMEM((CH, EMBEDDING_DIM), jnp.float32),
            pltpu.VMEM((CH, 128), jnp.float32),
            pltpu.VMEM((16,), jnp.float32),
            pltpu.SemaphoreType.DMA,
        ],
    )(_sc_lookup_kernel)
    w_pad = jnp.pad(W, ((0, 0), (0, 128 - EMBEDDING_DIM)))
    zq_st, loss_p = sc(z, w_pad, idx.reshape(N))

    loss, perp = pl.pallas_call(
        _finish_kernel,
        out_shape=(
            jax.ShapeDtypeStruct((1, 1), jnp.float32),
            jax.ShapeDtypeStruct((1, 1), jnp.float32),
        ),
    )(counts_p.reshape(GRID, NUM_EMBEDDINGS), loss_p)
    return (zq_st, loss[0, 0], (perp[0, 0], onehot, idx))


# speedup vs baseline: 1.0704x; 1.0704x over previous
import functools

import jax
import jax.numpy as jnp
from jax import lax
from jax.experimental import pallas as pl
from jax.experimental.pallas import tpu as pltpu
from jax.experimental.pallas import tpu_sc as plsc

NUM_EMBEDDINGS = 1024
EMBEDDING_DIM = 32
BETA = 0.25
N = 65536
BLOCK = 2048
GRID = N // BLOCK

NW = 32
ROWS_PER_W = N // NW
CH = 128
NCH = ROWS_PER_W // CH
SCH = 256


def _argmin_kernel(z_ref, w_ref, onehot_ref, idx_ref, counts_ref):
    z = z_ref[...]
    w = w_ref[...]

    zn = jnp.sum(z * z, axis=1, keepdims=True)
    wn = jnp.sum(w * w, axis=1)
    s = jnp.dot((z * -2.0).astype(jnp.bfloat16), w.astype(jnp.bfloat16).T,
                preferred_element_type=jnp.float32)
    dist = (zn + wn) + s

    iota = jax.lax.broadcasted_iota(
        jnp.int32, (BLOCK, NUM_EMBEDDINGS), 1).astype(jnp.float32)
    minval = jnp.min(dist, axis=1, keepdims=True)
    idxf = jnp.min(jnp.where(dist == minval, iota, float(NUM_EMBEDDINGS)),
                   axis=1, keepdims=True)
    onehot = (iota == idxf).astype(jnp.float32)
    onehot_ref[...] = onehot
    idx_ref[...] = idxf.astype(jnp.int32)
    ones_row = jnp.ones((1, BLOCK), dtype=jnp.bfloat16)
    counts_ref[...] = jnp.dot(ones_row, onehot.astype(jnp.bfloat16),
                              preferred_element_type=jnp.float32)[None]


def _bf16_round(x):
    c = x * 65537.0
    return c - (c - x)


def _sc_lookup_kernel(z_hbm, w_hbm, idx_hbm, zq_hbm, loss_hbm,
                      idx_v, zrows, grows, loss_v, sem):
    wid = lax.axis_index("s") * 2 + lax.axis_index("c")
    base_w = wid * ROWS_PER_W
    pltpu.sync_copy(idx_hbm.at[pl.ds(base_w, ROWS_PER_W)], idx_v)

    def chunk_body(c, acc):
        base = base_w + c * SCH
        copies = [
            pltpu.async_copy(
                w_hbm.at[idx_v.at[pl.ds(c * SCH + g * CH, CH)]],
                grows.at[pl.ds(g * CH, CH)], sem)
            for g in range(SCH // CH)
        ]
        pltpu.sync_copy(z_hbm.at[pl.ds(base, SCH)], zrows)
        for cp in copies:
            cp.wait()

        def row_body(r, a):
            for h in range(2):
                wv = grows[r, pl.ds(h * 16, 16)]
                zv = zrows[r, pl.ds(h * 16, 16)]
                zq = _bf16_round(wv)
                zrows[r, pl.ds(h * 16, 16)] = zv + (zq - zv)
                d = zq - zv
                a = a + d * d
            return a
        acc = lax.fori_loop(0, SCH, row_body, acc)

        pltpu.sync_copy(zrows, zq_hbm.at[pl.ds(base, SCH)])
        return acc

    loss_acc = lax.fori_loop(0, ROWS_PER_W // SCH, chunk_body,
                             jnp.zeros((16,), jnp.float32))
    loss_v[...] = loss_acc
    pltpu.sync_copy(loss_v, loss_hbm.at[wid])


def _finish_kernel(counts_ref, loss_ref, out_loss_ref, out_perp_ref):
    counts = jnp.sum(counts_ref[...], axis=0)
    e_mean = counts / float(N)
    out_perp_ref[...] = jnp.exp(
        -jnp.sum(e_mean * jnp.log(e_mean + 1e-10)))[None, None]
    loss_sum = jnp.sum(loss_ref[...])
    out_loss_ref[...] = (loss_sum * ((1.0 + BETA) / float(N * EMBEDDING_DIM))
                         )[None, None]


def kernel(z, W):
    onehot, idx, counts_p = pl.pallas_call(
        _argmin_kernel,
        grid=(GRID,),
        in_specs=[
            pl.BlockSpec((BLOCK, EMBEDDING_DIM), lambda i: (i, 0)),
            pl.BlockSpec((NUM_EMBEDDINGS, EMBEDDING_DIM), lambda i: (0, 0)),
        ],
        out_specs=(
            pl.BlockSpec((BLOCK, NUM_EMBEDDINGS), lambda i: (i, 0)),
            pl.BlockSpec((BLOCK, 1), lambda i: (i, 0)),
            pl.BlockSpec((1, 1, NUM_EMBEDDINGS), lambda i: (i, 0, 0)),
        ),
        out_shape=(
            jax.ShapeDtypeStruct((N, NUM_EMBEDDINGS), jnp.float32),
            jax.ShapeDtypeStruct((N, 1), jnp.int32),
            jax.ShapeDtypeStruct((GRID, 1, NUM_EMBEDDINGS), jnp.float32),
        ),
        compiler_params=pltpu.CompilerParams(
            dimension_semantics=("parallel",),
        ),
    )(z, W)

    sc = functools.partial(
        pl.kernel,
        out_type=(
            jax.ShapeDtypeStruct((N, EMBEDDING_DIM), jnp.float32),
            jax.ShapeDtypeStruct((NW, 16), jnp.float32),
        ),
        mesh=plsc.VectorSubcoreMesh(core_axis_name="c", subcore_axis_name="s"),
        scratch_types=[
            pltpu.VMEM((ROWS_PER_W,), jnp.int32),
            pltpu.VMEM((SCH, EMBEDDING_DIM), jnp.float32),
            pltpu.VMEM((SCH, 128), jnp.float32),
            pltpu.VMEM((16,), jnp.float32),
            pltpu.SemaphoreType.DMA,
        ],
    )(_sc_lookup_kernel)
    w_pad = jnp.pad(W, ((0, 0), (0, 128 - EMBEDDING_DIM)))
    zq_st, loss_p = sc(z, w_pad, idx.reshape(N))

    loss, perp = pl.pallas_call(
        _finish_kernel,
        out_shape=(
            jax.ShapeDtypeStruct((1, 1), jnp.float32),
            jax.ShapeDtypeStruct((1, 1), jnp.float32),
        ),
    )(counts_p.reshape(GRID, NUM_EMBEDDINGS), loss_p)
    return (zq_st, loss[0, 0], (perp[0, 0], onehot, idx))
